# Initial kernel scaffold; baseline (speedup 1.0000x reference)
import jax
import jax.numpy as jnp
from jax.experimental import pallas as pl


def kernel(x, edge_index, W, b):
    raise NotImplementedError("write your pallas kernel here")


# speedup vs baseline: 24.9665x; 24.9665x over previous
import dataclasses
import functools

import jax
import jax.numpy as jnp
from jax import lax
from jax.experimental import pallas as pl
from jax.experimental.pallas import tpu as pltpu
from jax.experimental.pallas import tpu_sc as plsc

N_NODES = 10000
N_PAD = 10240
N_EDGES = 320000
D = 128

NUM_CORES = 2
NUM_SUBCORES = 16
NUM_TILES = NUM_CORES * NUM_SUBCORES
EDGES_PER_TILE = N_EDGES // NUM_TILES
BATCH = 80
CHUNKS = EDGES_PER_TILE // BATCH
ROWS_PER_SUBCORE = N_PAD // NUM_SUBCORES
LANES = 16

_MESH = plsc.VectorSubcoreMesh(core_axis_name="c", subcore_axis_name="s")

_SC_PARAMS = pltpu.CompilerParams()
if "needs_layout_passes" in pltpu.CompilerParams.__dataclass_fields__:
    _SC_PARAMS = dataclasses.replace(_SC_PARAMS, needs_layout_passes=False)


@functools.partial(
    pl.kernel,
    out_type=jax.ShapeDtypeStruct((NUM_TILES, 1, N_PAD), jnp.float32),
    mesh=_MESH,
    scratch_types=[
        pltpu.VMEM((EDGES_PER_TILE,), jnp.int32),
        pltpu.VMEM((N_PAD,), jnp.float32),
    ],
    compiler_params=_SC_PARAMS,
)
def _degree_kernel(dst_hbm, out_hbm, idx_v, deg_v):
    c = lax.axis_index("c")
    s = lax.axis_index("s")
    wid = c * NUM_SUBCORES + s

    @pl.loop(0, N_PAD // LANES)
    def _(i):
        deg_v[pl.ds(i * LANES, LANES)] = jnp.zeros((LANES,), jnp.float32)

    pltpu.sync_copy(dst_hbm.at[wid, 0], idx_v)
    ones = jnp.full((LANES,), 1.0, jnp.float32)

    @pl.loop(0, EDGES_PER_TILE // LANES)
    def _(i):
        idx = idx_v[pl.ds(i * LANES, LANES)]
        plsc.addupdate_scatter(deg_v, [idx], ones)

    pltpu.sync_copy(deg_v, out_hbm.at[wid, 0])


@functools.partial(
    pl.kernel,
    out_type=jax.ShapeDtypeStruct((NUM_CORES, N_PAD, D), jnp.float32),
    mesh=_MESH,
    scratch_types=[
        pltpu.VMEM((CHUNKS, BATCH), jnp.int32),
        pltpu.VMEM((CHUNKS, BATCH), jnp.int32),
        pltpu.VMEM((BATCH, D), jnp.float32),
        pltpu.VMEM_SHARED((N_PAD, D), jnp.float32),
    ],
    compiler_params=_SC_PARAMS,
)
def _aggregate_kernel(g_hbm, src_hbm, dst_hbm, zero_hbm, out_hbm,
                      src_v, dst_v, rows_v, acc_shared):
    c = lax.axis_index("c")
    s = lax.axis_index("s")
    wid = c * NUM_SUBCORES + s
    row0 = s * ROWS_PER_SUBCORE

    pltpu.sync_copy(zero_hbm.at[pl.ds(row0, ROWS_PER_SUBCORE)],
                    acc_shared.at[pl.ds(row0, ROWS_PER_SUBCORE)])
    pltpu.sync_copy(src_hbm.at[wid], src_v)
    pltpu.sync_copy(dst_hbm.at[wid], dst_v)
    plsc.subcore_barrier()

    @pl.loop(0, CHUNKS)
    def _(j):
        pltpu.sync_copy(g_hbm.at[src_v.at[j]], rows_v)
        pltpu.sync_copy(rows_v, acc_shared.at[dst_v.at[j]], add=True)

    plsc.subcore_barrier()
    pltpu.sync_copy(acc_shared.at[pl.ds(row0, ROWS_PER_SUBCORE)],
                    out_hbm.at[c, pl.ds(row0, ROWS_PER_SUBCORE)])


_BLOCK = 2048


def _matmul_body(x_ref, w_ref, b_ref, h_ref):
    h_ref[...] = lax.dot_general(
        x_ref[...], w_ref[...], (((1,), (1,)), ((), ())),
        preferred_element_type=jnp.float32) + b_ref[...]


def _matmul(x, w, b2d):
    return pl.pallas_call(
        _matmul_body,
        grid=(N_PAD // _BLOCK,),
        in_specs=[
            pl.BlockSpec((_BLOCK, D), lambda i: (i, 0)),
            pl.BlockSpec((D, D), lambda i: (0, 0)),
            pl.BlockSpec((1, D), lambda i: (0, 0)),
        ],
        out_specs=pl.BlockSpec((_BLOCK, D), lambda i: (i, 0)),
        out_shape=jax.ShapeDtypeStruct((N_PAD, D), jnp.float32),
    )(x, w, b2d)


def _scale_body(pd_ref, h_ref, g_ref, r_ref):
    deg = jnp.sum(pd_ref[...], axis=1, keepdims=True) + 1.0
    r = lax.rsqrt(deg)
    r_ref[...] = r
    g_ref[...] = h_ref[...] * r


def _scale(pd_t, h):
    return pl.pallas_call(
        _scale_body,
        grid=(N_PAD // _BLOCK,),
        in_specs=[
            pl.BlockSpec((_BLOCK, NUM_TILES), lambda i: (i, 0)),
            pl.BlockSpec((_BLOCK, D), lambda i: (i, 0)),
        ],
        out_specs=[
            pl.BlockSpec((_BLOCK, D), lambda i: (i, 0)),
            pl.BlockSpec((_BLOCK, 1), lambda i: (i, 0)),
        ],
        out_shape=[
            jax.ShapeDtypeStruct((N_PAD, D), jnp.float32),
            jax.ShapeDtypeStruct((N_PAD, 1), jnp.float32),
        ],
    )(pd_t, h)


def _combine_body(p_ref, g_ref, r_ref, o_ref):
    o_ref[...] = (p_ref[0] + p_ref[1] + g_ref[...]) * r_ref[...]


def _combine(partials, g, r):
    return pl.pallas_call(
        _combine_body,
        grid=(N_PAD // _BLOCK,),
        in_specs=[
            pl.BlockSpec((NUM_CORES, _BLOCK, D), lambda i: (0, i, 0)),
            pl.BlockSpec((_BLOCK, D), lambda i: (i, 0)),
            pl.BlockSpec((_BLOCK, 1), lambda i: (i, 0)),
        ],
        out_specs=pl.BlockSpec((_BLOCK, D), lambda i: (i, 0)),
        out_shape=jax.ShapeDtypeStruct((N_PAD, D), jnp.float32),
    )(partials, g, r)


def kernel(x, edge_index, W, b):
    src = edge_index[0].astype(jnp.int32)
    dst = edge_index[1].astype(jnp.int32)
    dst_tiles = dst.reshape(NUM_TILES, 1, EDGES_PER_TILE)
    src_chunks = src.reshape(NUM_TILES, CHUNKS, BATCH)
    dst_chunks = dst.reshape(NUM_TILES, CHUNKS, BATCH)
    xp = jnp.pad(x, ((0, N_PAD - N_NODES), (0, 0)))

    partial_deg = _degree_kernel(dst_tiles)
    h = _matmul(xp, W, b.reshape(1, D))
    g, r = _scale(partial_deg.reshape(NUM_TILES, N_PAD).T, h)
    zeros = jnp.zeros((N_PAD, D), jnp.float32)
    partials = _aggregate_kernel(g, src_chunks, dst_chunks, zeros)
    out = _combine(partials, g, r)
    return out[:N_NODES]
